# Initial kernel scaffold; baseline (speedup 1.0000x reference)
import jax
import jax.numpy as jnp
from jax.experimental import pallas as pl


def kernel(x, edge_index, W1, a_src1, a_dst1, b1, W2, a_src2, a_dst2, b2):
    raise NotImplementedError("write your pallas kernel here")


# speedup vs baseline: 18.2408x; 18.2408x over previous
import jax
import jax.numpy as jnp
from jax import lax
from jax.experimental import pallas as pl
from jax.experimental.pallas import tpu as pltpu
from jax.experimental.pallas import tpu_sc as plsc

N = 10000
E = 320000
D = 128
NPAD = 10240
NC = 2
NS = 16
NW = NC * NS
CH = 128
NCHW = 80
EPAD = NW * NCHW * CH
RPS = NPAD // NS
BLK = 256
NBLK = NPAD // BLK

_mesh = plsc.VectorSubcoreMesh(
    core_axis_name="c", subcore_axis_name="s", num_cores=NC, num_subcores=NS)


def _lrelu(v):
    return jnp.where(v >= 0.0, v, 0.2 * v)


def _tc1_body(x_ref, w_ref, asrc_ref, adst_ref,
              h_ref, as_ref, ad_ref, g_ref, m_ref):
    i = pl.program_id(0)
    h = jnp.dot(x_ref[...], w_ref[...], preferred_element_type=jnp.float32)
    h_ref[...] = h
    a1 = h @ asrc_ref[...]
    a2 = h @ adst_ref[...]
    as_ref[...] = a1
    ad_ref[...] = a2
    bm1 = jnp.max(a1)
    bm2 = jnp.max(a2)

    @pl.when(i == 0)
    def _():
        m_ref[0] = bm1
        m_ref[1] = bm2

    @pl.when(i > 0)
    def _():
        m_ref[0] = jnp.maximum(m_ref[0], bm1)
        m_ref[1] = jnp.maximum(m_ref[1], bm2)

    z = m_ref[0] + m_ref[1]
    g = jnp.where(z >= 0.0, z, 0.2 * z)
    g_ref[...] = jnp.full((8, 128), g, jnp.float32)


_tc1 = pl.pallas_call(
    _tc1_body,
    grid=(NBLK,),
    in_specs=[
        pl.BlockSpec((BLK, D), lambda i: (i, 0)),
        pl.BlockSpec((D, D), lambda i: (0, 0)),
        pl.BlockSpec((D,), lambda i: (0,)),
        pl.BlockSpec((D,), lambda i: (0,)),
    ],
    out_specs=[
        pl.BlockSpec((BLK, D), lambda i: (i, 0)),
        pl.BlockSpec((BLK,), lambda i: (i,)),
        pl.BlockSpec((BLK,), lambda i: (i,)),
        pl.BlockSpec((8, 128), lambda i: (0, 0)),
    ],
    out_shape=[
        jax.ShapeDtypeStruct((NPAD, D), jnp.float32),
        jax.ShapeDtypeStruct((NPAD,), jnp.float32),
        jax.ShapeDtypeStruct((NPAD,), jnp.float32),
        jax.ShapeDtypeStruct((8, 128), jnp.float32),
    ],
    scratch_shapes=[pltpu.SMEM((2,), jnp.float32)],
)


def _tc2_body(p_ref, b1_ref, w0_ref, w1_ref, was_ref, wad_ref,
              hc_ref, as_ref, ad_ref, g_ref, m_ref):
    i = pl.program_id(0)
    hin = jax.nn.relu(p_ref[0] + p_ref[1] + b1_ref[...])
    c0 = hin @ w0_ref[...]
    c1 = hin @ w1_ref[...]
    hc_ref[0, :] = c0
    hc_ref[1, :] = c1
    a1 = hin @ was_ref[...]
    a2 = hin @ wad_ref[...]
    as_ref[...] = a1
    ad_ref[...] = a2
    bm1 = jnp.max(a1)
    bm2 = jnp.max(a2)

    @pl.when(i == 0)
    def _():
        m_ref[0] = bm1
        m_ref[1] = bm2

    @pl.when(i > 0)
    def _():
        m_ref[0] = jnp.maximum(m_ref[0], bm1)
        m_ref[1] = jnp.maximum(m_ref[1], bm2)

    z = m_ref[0] + m_ref[1]
    g = jnp.where(z >= 0.0, z, 0.2 * z)
    g_ref[...] = jnp.full((8, 128), g, jnp.float32)


_tc2 = pl.pallas_call(
    _tc2_body,
    grid=(NBLK,),
    in_specs=[
        pl.BlockSpec((2, BLK, D), lambda i: (0, i, 0)),
        pl.BlockSpec((D,), lambda i: (0,)),
        pl.BlockSpec((D,), lambda i: (0,)),
        pl.BlockSpec((D,), lambda i: (0,)),
        pl.BlockSpec((D,), lambda i: (0,)),
        pl.BlockSpec((D,), lambda i: (0,)),
    ],
    out_specs=[
        pl.BlockSpec((2, BLK), lambda i: (0, i)),
        pl.BlockSpec((BLK,), lambda i: (i,)),
        pl.BlockSpec((BLK,), lambda i: (i,)),
        pl.BlockSpec((8, 128), lambda i: (0, 0)),
    ],
    out_shape=[
        jax.ShapeDtypeStruct((2, NPAD), jnp.float32),
        jax.ShapeDtypeStruct((NPAD,), jnp.float32),
        jax.ShapeDtypeStruct((NPAD,), jnp.float32),
        jax.ShapeDtypeStruct((8, 128), jnp.float32),
    ],
    scratch_shapes=[pltpu.SMEM((2,), jnp.float32)],
)


def _edge_body(src_hbm, dst_hbm, as_hbm, ad_hbm, g_hbm,
               ex_hbm, sp_hbm,
               src_v, dst_v, asg, adg, ex_v, gbuf, zer, s_sh, sem, sem2):
    cid = lax.axis_index("c")
    sid = lax.axis_index("s")
    wid = sid * NC + cid
    wbase = wid * NCHW
    pltpu.sync_copy(src_hbm.at[pl.ds(wbase, NCHW)], src_v)
    pltpu.sync_copy(dst_hbm.at[pl.ds(wbase, NCHW)], dst_v)
    pltpu.sync_copy(g_hbm, gbuf)

    def zb(i, carry):
        zer[pl.ds(i * 16, 16)] = jnp.zeros((16,), jnp.float32)
        return carry
    lax.fori_loop(0, RPS // 16, zb, 0)
    pltpu.sync_copy(zer, s_sh.at[pl.ds(sid * RPS, RPS)])
    plsc.subcore_barrier()

    gl = gbuf[...]

    def body(j, carry):
        cpa = pltpu.async_copy(as_hbm.at[src_v.at[j]], asg, sem)
        cpb = pltpu.async_copy(ad_hbm.at[dst_v.at[j]], adg, sem2)
        cpa.wait()
        cpb.wait()
        for c in range(CH // 16):
            sl = pl.ds(c * 16, 16)
            v = asg[sl] + adg[sl]
            ex_v[j, sl] = jnp.exp(_lrelu(v) - gl)
        pltpu.sync_copy(ex_v.at[j], s_sh.at[dst_v.at[j]], add=True)
        return carry
    lax.fori_loop(0, NCHW, body, 0)

    pltpu.sync_copy(ex_v, ex_hbm.at[pl.ds(wbase, NCHW)])
    plsc.subcore_barrier()
    pltpu.sync_copy(s_sh.at[pl.ds(sid * RPS, RPS)],
                    sp_hbm.at[cid, pl.ds(sid * RPS, RPS)])


_edge_softmax = pl.kernel(
    _edge_body,
    out_type=[
        jax.ShapeDtypeStruct((NW * NCHW, CH), jnp.float32),
        jax.ShapeDtypeStruct((NC, NPAD), jnp.float32),
    ],
    mesh=_mesh,
    scratch_types=[
        pltpu.VMEM((NCHW, CH), jnp.int32),
        pltpu.VMEM((NCHW, CH), jnp.int32),
        pltpu.VMEM((CH,), jnp.float32),
        pltpu.VMEM((CH,), jnp.float32),
        pltpu.VMEM((NCHW, CH), jnp.float32),
        pltpu.VMEM((16,), jnp.float32),
        pltpu.VMEM((RPS,), jnp.float32),
        pltpu.VMEM_SHARED((NPAD,), jnp.float32),
        pltpu.SemaphoreType.DMA,
        pltpu.SemaphoreType.DMA,
    ],
)


def _agg1_body(src_hbm, dst_hbm, ex_hbm, sp0_hbm, sp1_hbm, h_hbm,
               op_hbm,
               src_v, dst_v, ex_v, sv0, sv1, coefrow, rows,
               out_sh, sem, sem2, sem3):
    cid = lax.axis_index("c")
    sid = lax.axis_index("s")
    wid = sid * NC + cid
    wbase = wid * NCHW
    pltpu.sync_copy(src_hbm.at[pl.ds(wbase, NCHW)], src_v)
    pltpu.sync_copy(dst_hbm.at[pl.ds(wbase, NCHW)], dst_v)
    pltpu.sync_copy(ex_hbm.at[pl.ds(wbase, NCHW)], ex_v)

    def zb(r, carry):
        for c in range(D // 16):
            rows[r, pl.ds(c * 16, 16)] = jnp.zeros((16,), jnp.float32)
        return carry
    lax.fori_loop(0, CH, zb, 0)
    for b in range(RPS // CH):
        pltpu.sync_copy(rows, out_sh.at[pl.ds(sid * RPS + b * CH, CH)])
    plsc.subcore_barrier()

    def body(j, carry):
        cpr = pltpu.async_copy(h_hbm.at[src_v.at[j]], rows, sem)
        cp0 = pltpu.async_copy(sp0_hbm.at[dst_v.at[j]], sv0, sem2)
        cp1 = pltpu.async_copy(sp1_hbm.at[dst_v.at[j]], sv1, sem3)
        cp0.wait()
        cp1.wait()
        for c in range(CH // 16):
            sl = pl.ds(c * 16, 16)
            coefrow[sl] = ex_v[j, sl] / (sv0[sl] + sv1[sl] + 1e-16)
        cpr.wait()

        def sc(g, carry2):
            cv = coefrow[pl.ds(g * 16, 16)]
            for k in range(16):
                cb = lax.gather(
                    cv, jnp.full((16, 1), k, jnp.int32),
                    lax.GatherDimensionNumbers(
                        offset_dims=(), collapsed_slice_dims=(0,),
                        start_index_map=(0,)),
                    (1,), mode=lax.GatherScatterMode.PROMISE_IN_BOUNDS)
                r = g * 16 + k
                for c in range(D // 16):
                    sl = pl.ds(c * 16, 16)
                    rows[r, sl] = rows[r, sl] * cb
            return carry2
        lax.fori_loop(0, CH // 16, sc, 0)
        pltpu.sync_copy(rows, out_sh.at[dst_v.at[j]], add=True)
        return carry
    lax.fori_loop(0, NCHW, body, 0)

    plsc.subcore_barrier()
    for b in range(RPS // CH):
        r0 = sid * RPS + b * CH
        pltpu.sync_copy(out_sh.at[pl.ds(r0, CH)],
                        op_hbm.at[cid, pl.ds(r0, CH)])


_agg1 = pl.kernel(
    _agg1_body,
    out_type=[
        jax.ShapeDtypeStruct((NC, NPAD, D), jnp.float32),
    ],
    mesh=_mesh,
    scratch_types=[
        pltpu.VMEM((NCHW, CH), jnp.int32),
        pltpu.VMEM((NCHW, CH), jnp.int32),
        pltpu.VMEM((NCHW, CH), jnp.float32),
        pltpu.VMEM((CH,), jnp.float32),
        pltpu.VMEM((CH,), jnp.float32),
        pltpu.VMEM((CH,), jnp.float32),
        pltpu.VMEM((CH, D), jnp.float32),
        pltpu.VMEM_SHARED((NPAD, D), jnp.float32),
        pltpu.SemaphoreType.DMA,
        pltpu.SemaphoreType.DMA,
        pltpu.SemaphoreType.DMA,
    ],
)


def _agg2_body(src_hbm, dst_hbm, ex_hbm, sp0_hbm, sp1_hbm, c0_hbm, c1_hbm,
               op_hbm,
               src_v, dst_v, ex_v, sv0, sv1, c0g, c1g, y0, y1, zer,
               s0_sh, s1_sh, sem, sem2, sem3, sem4):
    cid = lax.axis_index("c")
    sid = lax.axis_index("s")
    wid = sid * NC + cid
    wbase = wid * NCHW
    pltpu.sync_copy(src_hbm.at[pl.ds(wbase, NCHW)], src_v)
    pltpu.sync_copy(dst_hbm.at[pl.ds(wbase, NCHW)], dst_v)
    pltpu.sync_copy(ex_hbm.at[pl.ds(wbase, NCHW)], ex_v)

    def zb(i, carry):
        zer[pl.ds(i * 16, 16)] = jnp.zeros((16,), jnp.float32)
        return carry
    lax.fori_loop(0, RPS // 16, zb, 0)
    pltpu.sync_copy(zer, s0_sh.at[pl.ds(sid * RPS, RPS)])
    pltpu.sync_copy(zer, s1_sh.at[pl.ds(sid * RPS, RPS)])
    plsc.subcore_barrier()

    def body(j, carry):
        cpa = pltpu.async_copy(c0_hbm.at[src_v.at[j]], c0g, sem)
        cpb = pltpu.async_copy(c1_hbm.at[src_v.at[j]], c1g, sem2)
        cp0 = pltpu.async_copy(sp0_hbm.at[dst_v.at[j]], sv0, sem3)
        cp1 = pltpu.async_copy(sp1_hbm.at[dst_v.at[j]], sv1, sem4)
        cpa.wait()
        cpb.wait()
        cp0.wait()
        cp1.wait()
        for c in range(CH // 16):
            sl = pl.ds(c * 16, 16)
            coef = ex_v[j, sl] / (sv0[sl] + sv1[sl] + 1e-16)
            y0[sl] = coef * c0g[sl]
            y1[sl] = coef * c1g[sl]
        pltpu.sync_copy(y0, s0_sh.at[dst_v.at[j]], add=True)
        pltpu.sync_copy(y1, s1_sh.at[dst_v.at[j]], add=True)
        return carry
    lax.fori_loop(0, NCHW, body, 0)

    plsc.subcore_barrier()
    pltpu.sync_copy(s0_sh.at[pl.ds(sid * RPS, RPS)],
                    op_hbm.at[cid, 0, pl.ds(sid * RPS, RPS)])
    pltpu.sync_copy(s1_sh.at[pl.ds(sid * RPS, RPS)],
                    op_hbm.at[cid, 1, pl.ds(sid * RPS, RPS)])


_agg2 = pl.kernel(
    _agg2_body,
    out_type=[
        jax.ShapeDtypeStruct((NC, 2, NPAD), jnp.float32),
    ],
    mesh=_mesh,
    scratch_types=[
        pltpu.VMEM((NCHW, CH), jnp.int32),
        pltpu.VMEM((NCHW, CH), jnp.int32),
        pltpu.VMEM((NCHW, CH), jnp.float32),
        pltpu.VMEM((CH,), jnp.float32),
        pltpu.VMEM((CH,), jnp.float32),
        pltpu.VMEM((CH,), jnp.float32),
        pltpu.VMEM((CH,), jnp.float32),
        pltpu.VMEM((CH,), jnp.float32),
        pltpu.VMEM((CH,), jnp.float32),
        pltpu.VMEM((RPS,), jnp.float32),
        pltpu.VMEM_SHARED((NPAD,), jnp.float32),
        pltpu.VMEM_SHARED((NPAD,), jnp.float32),
        pltpu.SemaphoreType.DMA,
        pltpu.SemaphoreType.DMA,
        pltpu.SemaphoreType.DMA,
        pltpu.SemaphoreType.DMA,
    ],
)


def kernel(x, edge_index, W1, a_src1, a_dst1, b1, W2, a_src2, a_dst2, b2):
    xp = jnp.pad(x, ((0, NPAD - N), (0, 0)))
    src = edge_index[0].astype(jnp.int32)
    dst = edge_index[1].astype(jnp.int32)
    padv = jnp.full((EPAD - E,), NPAD - 1, jnp.int32)
    src2d = jnp.concatenate([src, padv]).reshape(NW * NCHW, CH)
    dst2d = jnp.concatenate([dst, padv]).reshape(NW * NCHW, CH)

    h1, as1, ad1, g1 = _tc1(xp, W1, a_src1, a_dst1)
    ex1, s1p = _edge_softmax(src2d, dst2d, as1, ad1, g1[0, :16])
    (o1p,) = _agg1(src2d, dst2d, ex1, s1p[0], s1p[1], h1)

    w2c0 = W2[:, 0]
    w2c1 = W2[:, 1]
    was2 = W2 @ a_src2
    wad2 = W2 @ a_dst2
    h2c, as2, ad2, g2 = _tc2(o1p, b1, w2c0, w2c1, was2, wad2)
    ex2, s2p = _edge_softmax(src2d, dst2d, as2, ad2, g2[0, :16])
    (o2p,) = _agg2(src2d, dst2d, ex2, s2p[0], s2p[1], h2c[0], h2c[1])

    o2 = o2p[0] + o2p[1]
    return o2[:, :N].T + b2


# speedup vs baseline: 20.6912x; 1.1343x over previous
import jax
import jax.numpy as jnp
from jax import lax
from jax.experimental import pallas as pl
from jax.experimental.pallas import tpu as pltpu
from jax.experimental.pallas import tpu_sc as plsc

N = 10000
E = 320000
D = 128
NPAD = 10240
NC = 2
NS = 16
NW = NC * NS
CH = 128
NCHW = 80
EPAD = NW * NCHW * CH
RPS = NPAD // NS
BLK = 256
NBLK = NPAD // BLK

_mesh = plsc.VectorSubcoreMesh(
    core_axis_name="c", subcore_axis_name="s", num_cores=NC, num_subcores=NS)


def _lrelu(v):
    return jnp.where(v >= 0.0, v, 0.2 * v)


def _tc1_body(x_ref, w_ref, asrc_ref, adst_ref,
              h_ref, as_ref, ad_ref, g_ref, m_ref):
    i = pl.program_id(0)
    h = jnp.dot(x_ref[...], w_ref[...], preferred_element_type=jnp.float32)
    h_ref[...] = h
    a1 = h @ asrc_ref[...]
    a2 = h @ adst_ref[...]
    as_ref[...] = a1
    ad_ref[...] = a2
    bm1 = jnp.max(a1)
    bm2 = jnp.max(a2)

    @pl.when(i == 0)
    def _():
        m_ref[0] = bm1
        m_ref[1] = bm2

    @pl.when(i > 0)
    def _():
        m_ref[0] = jnp.maximum(m_ref[0], bm1)
        m_ref[1] = jnp.maximum(m_ref[1], bm2)

    z = m_ref[0] + m_ref[1]
    g = jnp.where(z >= 0.0, z, 0.2 * z)
    g_ref[...] = jnp.full((8, 128), g, jnp.float32)


_tc1 = pl.pallas_call(
    _tc1_body,
    grid=(NBLK,),
    in_specs=[
        pl.BlockSpec((BLK, D), lambda i: (i, 0)),
        pl.BlockSpec((D, D), lambda i: (0, 0)),
        pl.BlockSpec((D,), lambda i: (0,)),
        pl.BlockSpec((D,), lambda i: (0,)),
    ],
    out_specs=[
        pl.BlockSpec((BLK, D), lambda i: (i, 0)),
        pl.BlockSpec((BLK,), lambda i: (i,)),
        pl.BlockSpec((BLK,), lambda i: (i,)),
        pl.BlockSpec((8, 128), lambda i: (0, 0)),
    ],
    out_shape=[
        jax.ShapeDtypeStruct((NPAD, D), jnp.float32),
        jax.ShapeDtypeStruct((NPAD,), jnp.float32),
        jax.ShapeDtypeStruct((NPAD,), jnp.float32),
        jax.ShapeDtypeStruct((8, 128), jnp.float32),
    ],
    scratch_shapes=[pltpu.SMEM((2,), jnp.float32)],
)


def _tc2_body(pa_ref, pb_ref, b1_ref, w0_ref, w1_ref, was_ref, wad_ref,
              hc_ref, as_ref, ad_ref, g_ref, m_ref):
    i = pl.program_id(0)
    hfull = jnp.concatenate([pa_ref[0] + pa_ref[1], pb_ref[0] + pb_ref[1]],
                            axis=1)
    hin = jax.nn.relu(hfull + b1_ref[...])
    c0 = hin @ w0_ref[...]
    c1 = hin @ w1_ref[...]
    hc_ref[0, :] = c0
    hc_ref[1, :] = c1
    a1 = hin @ was_ref[...]
    a2 = hin @ wad_ref[...]
    as_ref[...] = a1
    ad_ref[...] = a2
    bm1 = jnp.max(a1)
    bm2 = jnp.max(a2)

    @pl.when(i == 0)
    def _():
        m_ref[0] = bm1
        m_ref[1] = bm2

    @pl.when(i > 0)
    def _():
        m_ref[0] = jnp.maximum(m_ref[0], bm1)
        m_ref[1] = jnp.maximum(m_ref[1], bm2)

    z = m_ref[0] + m_ref[1]
    g = jnp.where(z >= 0.0, z, 0.2 * z)
    g_ref[...] = jnp.full((8, 128), g, jnp.float32)


_tc2 = pl.pallas_call(
    _tc2_body,
    grid=(NBLK,),
    in_specs=[
        pl.BlockSpec((2, BLK, D // 2), lambda i: (0, i, 0)),
        pl.BlockSpec((2, BLK, D // 2), lambda i: (0, i, 0)),
        pl.BlockSpec((D,), lambda i: (0,)),
        pl.BlockSpec((D,), lambda i: (0,)),
        pl.BlockSpec((D,), lambda i: (0,)),
        pl.BlockSpec((D,), lambda i: (0,)),
        pl.BlockSpec((D,), lambda i: (0,)),
    ],
    out_specs=[
        pl.BlockSpec((2, BLK), lambda i: (0, i)),
        pl.BlockSpec((BLK,), lambda i: (i,)),
        pl.BlockSpec((BLK,), lambda i: (i,)),
        pl.BlockSpec((8, 128), lambda i: (0, 0)),
    ],
    out_shape=[
        jax.ShapeDtypeStruct((2, NPAD), jnp.float32),
        jax.ShapeDtypeStruct((NPAD,), jnp.float32),
        jax.ShapeDtypeStruct((NPAD,), jnp.float32),
        jax.ShapeDtypeStruct((8, 128), jnp.float32),
    ],
    scratch_shapes=[pltpu.SMEM((2,), jnp.float32)],
)


def _edge_body(src_hbm, dst_hbm, as_hbm, ad_hbm, g_hbm,
               ex_hbm, sp_hbm,
               src_v, dst_v, asg, adg, ex_v, gbuf, zer, s_sh, sem, sem2):
    cid = lax.axis_index("c")
    sid = lax.axis_index("s")
    wid = sid * NC + cid
    wbase = wid * NCHW
    pltpu.sync_copy(src_hbm.at[pl.ds(wbase, NCHW)], src_v)
    pltpu.sync_copy(dst_hbm.at[pl.ds(wbase, NCHW)], dst_v)
    pltpu.sync_copy(g_hbm, gbuf)

    def zb(i, carry):
        zer[pl.ds(i * 16, 16)] = jnp.zeros((16,), jnp.float32)
        return carry
    lax.fori_loop(0, RPS // 16, zb, 0)
    pltpu.sync_copy(zer, s_sh.at[pl.ds(sid * RPS, RPS)])
    plsc.subcore_barrier()

    gl = gbuf[...]

    def body(j, carry):
        cpa = pltpu.async_copy(as_hbm.at[src_v.at[j]], asg, sem)
        cpb = pltpu.async_copy(ad_hbm.at[dst_v.at[j]], adg, sem2)
        cpa.wait()
        cpb.wait()
        for c in range(CH // 16):
            sl = pl.ds(c * 16, 16)
            v = asg[sl] + adg[sl]
            ex_v[j, sl] = jnp.exp(_lrelu(v) - gl)
        pltpu.sync_copy(ex_v.at[j], s_sh.at[dst_v.at[j]], add=True)
        return carry
    lax.fori_loop(0, NCHW, body, 0)

    pltpu.sync_copy(ex_v, ex_hbm.at[pl.ds(wbase, NCHW)])
    plsc.subcore_barrier()
    pltpu.sync_copy(s_sh.at[pl.ds(sid * RPS, RPS)],
                    sp_hbm.at[cid, pl.ds(sid * RPS, RPS)])


_edge_softmax = pl.kernel(
    _edge_body,
    out_type=[
        jax.ShapeDtypeStruct((NW * NCHW, CH), jnp.float32),
        jax.ShapeDtypeStruct((NC, NPAD), jnp.float32),
    ],
    mesh=_mesh,
    scratch_types=[
        pltpu.VMEM((NCHW, CH), jnp.int32),
        pltpu.VMEM((NCHW, CH), jnp.int32),
        pltpu.VMEM((CH,), jnp.float32),
        pltpu.VMEM((CH,), jnp.float32),
        pltpu.VMEM((NCHW, CH), jnp.float32),
        pltpu.VMEM((16,), jnp.float32),
        pltpu.VMEM((RPS,), jnp.float32),
        pltpu.VMEM_SHARED((NPAD,), jnp.float32),
        pltpu.SemaphoreType.DMA,
        pltpu.SemaphoreType.DMA,
    ],
)


NBUF = 4
DH = D // 2


def _agg1_body(src_hbm, dst_hbm, ex_hbm, sp0_hbm, sp1_hbm, h_hbm,
               op_hbm,
               src_v, dst_v, ex_v, coefrow, rows4, sv0_4, sv1_4,
               out_sh, gs0, gs1, gs2, gs3, ss0, ss1, ss2, ss3):
    gsem = (gs0, gs1, gs2, gs3)
    ssem = (ss0, ss1, ss2, ss3)
    cid = lax.axis_index("c")
    sid = lax.axis_index("s")
    wid = sid * NC + cid
    wbase = wid * NCHW
    pltpu.sync_copy(src_hbm.at[pl.ds(wbase, NCHW)], src_v)
    pltpu.sync_copy(dst_hbm.at[pl.ds(wbase, NCHW)], dst_v)
    pltpu.sync_copy(ex_hbm.at[pl.ds(wbase, NCHW)], ex_v)

    rows0 = rows4.at[0]

    def zb(r, carry):
        for c in range(DH // 16):
            rows4[0, r, pl.ds(c * 16, 16)] = jnp.zeros((16,), jnp.float32)
        return carry
    lax.fori_loop(0, CH, zb, 0)
    for b in range(RPS // CH):
        pltpu.sync_copy(rows0, out_sh.at[pl.ds(sid * RPS + b * CH, CH)])
    plsc.subcore_barrier()

    def gstart(j, b):
        pltpu.async_copy(h_hbm.at[src_v.at[j]], rows4.at[b], gsem[b])
        pltpu.async_copy(sp0_hbm.at[dst_v.at[j]], sv0_4.at[b], gsem[b])
        pltpu.async_copy(sp1_hbm.at[dst_v.at[j]], sv1_4.at[b], gsem[b])

    def gwait(j, b):
        pltpu.make_async_copy(h_hbm.at[src_v.at[j]], rows4.at[b],
                              gsem[b]).wait()
        pltpu.make_async_copy(sp0_hbm.at[dst_v.at[j]], sv0_4.at[b],
                              gsem[b]).wait()
        pltpu.make_async_copy(sp1_hbm.at[dst_v.at[j]], sv1_4.at[b],
                              gsem[b]).wait()

    def sstart(j, b):
        pltpu.async_copy(rows4.at[b], out_sh.at[dst_v.at[j]], ssem[b],
                         add=True)

    def swait(j, b):
        pltpu.make_async_copy(rows4.at[b], out_sh.at[dst_v.at[j]],
                              ssem[b]).wait()

    def compute(j, b):
        for c in range(CH // 16):
            sl = pl.ds(c * 16, 16)
            coefrow[sl] = ex_v[j, sl] / (sv0_4[b, sl] + sv1_4[b, sl] + 1e-16)

        def sc(g, carry2):
            cv = coefrow[pl.ds(g * 16, 16)]
            for k in range(16):
                cb = lax.gather(
                    cv, jnp.full((16, 1), k, jnp.int32),
                    lax.GatherDimensionNumbers(
                        offset_dims=(), collapsed_slice_dims=(0,),
                        start_index_map=(0,)),
                    (1,), mode=lax.GatherScatterMode.PROMISE_IN_BOUNDS)
                r = g * 16 + k
                for c in range(DH // 16):
                    sl = pl.ds(c * 16, 16)
                    rows4[b, r, sl] = rows4[b, r, sl] * cb
            return carry2
        lax.fori_loop(0, CH // 16, sc, 0)

    gstart(0, 0)
    gstart(1, 1)

    def body(i, carry):
        for b in range(NBUF):
            j = i * NBUF + b
            gwait(j, b)
            compute(j, b)
            bn = (b + 2) % NBUF
            if b in (0, 1):
                @pl.when(i > 0)
                def _():
                    swait(j - 2, bn)
                gstart(j + 2, bn)
            else:
                swait(j - 2, bn)

                @pl.when(i < NCHW // NBUF - 1)
                def _():
                    gstart(j + 2, bn)
            sstart(j, b)
        return carry
    lax.fori_loop(0, NCHW // NBUF, body, 0)

    swait(NCHW - 2, (NCHW - 2) % NBUF)
    swait(NCHW - 1, (NCHW - 1) % NBUF)

    plsc.subcore_barrier()
    for b in range(RPS // CH):
        r0 = sid * RPS + b * CH
        pltpu.sync_copy(out_sh.at[pl.ds(r0, CH)],
                        op_hbm.at[cid, pl.ds(r0, CH)])


_agg1 = pl.kernel(
    _agg1_body,
    out_type=[
        jax.ShapeDtypeStruct((NC, NPAD, DH), jnp.float32),
    ],
    mesh=_mesh,
    compiler_params=pltpu.CompilerParams(use_tc_tiling_on_sc=False),
    scratch_types=[
        pltpu.VMEM((NCHW, CH), jnp.int32),
        pltpu.VMEM((NCHW, CH), jnp.int32),
        pltpu.VMEM((NCHW, CH), jnp.float32),
        pltpu.VMEM((CH,), jnp.float32),
        pltpu.VMEM((NBUF, CH, DH), jnp.float32),
        pltpu.VMEM((NBUF, CH), jnp.float32),
        pltpu.VMEM((NBUF, CH), jnp.float32),
        pltpu.VMEM_SHARED((NPAD, DH), jnp.float32),
        pltpu.SemaphoreType.DMA,
        pltpu.SemaphoreType.DMA,
        pltpu.SemaphoreType.DMA,
        pltpu.SemaphoreType.DMA,
        pltpu.SemaphoreType.DMA,
        pltpu.SemaphoreType.DMA,
        pltpu.SemaphoreType.DMA,
        pltpu.SemaphoreType.DMA,
    ],
)


def _agg2_body(src_hbm, dst_hbm, ex_hbm, sp0_hbm, sp1_hbm, c0_hbm, c1_hbm,
               op_hbm,
               src_v, dst_v, ex_v, sv0, sv1, c0g, c1g, y0, y1, zer,
               s0_sh, s1_sh, sem, sem2, sem3, sem4):
    cid = lax.axis_index("c")
    sid = lax.axis_index("s")
    wid = sid * NC + cid
    wbase = wid * NCHW
    pltpu.sync_copy(src_hbm.at[pl.ds(wbase, NCHW)], src_v)
    pltpu.sync_copy(dst_hbm.at[pl.ds(wbase, NCHW)], dst_v)
    pltpu.sync_copy(ex_hbm.at[pl.ds(wbase, NCHW)], ex_v)

    def zb(i, carry):
        zer[pl.ds(i * 16, 16)] = jnp.zeros((16,), jnp.float32)
        return carry
    lax.fori_loop(0, RPS // 16, zb, 0)
    pltpu.sync_copy(zer, s0_sh.at[pl.ds(sid * RPS, RPS)])
    pltpu.sync_copy(zer, s1_sh.at[pl.ds(sid * RPS, RPS)])
    plsc.subcore_barrier()

    def body(j, carry):
        cpa = pltpu.async_copy(c0_hbm.at[src_v.at[j]], c0g, sem)
        cpb = pltpu.async_copy(c1_hbm.at[src_v.at[j]], c1g, sem2)
        cp0 = pltpu.async_copy(sp0_hbm.at[dst_v.at[j]], sv0, sem3)
        cp1 = pltpu.async_copy(sp1_hbm.at[dst_v.at[j]], sv1, sem4)
        cpa.wait()
        cpb.wait()
        cp0.wait()
        cp1.wait()
        for c in range(CH // 16):
            sl = pl.ds(c * 16, 16)
            coef = ex_v[j, sl] / (sv0[sl] + sv1[sl] + 1e-16)
            y0[sl] = coef * c0g[sl]
            y1[sl] = coef * c1g[sl]
        pltpu.sync_copy(y0, s0_sh.at[dst_v.at[j]], add=True)
        pltpu.sync_copy(y1, s1_sh.at[dst_v.at[j]], add=True)
        return carry
    lax.fori_loop(0, NCHW, body, 0)

    plsc.subcore_barrier()
    pltpu.sync_copy(s0_sh.at[pl.ds(sid * RPS, RPS)],
                    op_hbm.at[cid, 0, pl.ds(sid * RPS, RPS)])
    pltpu.sync_copy(s1_sh.at[pl.ds(sid * RPS, RPS)],
                    op_hbm.at[cid, 1, pl.ds(sid * RPS, RPS)])


_agg2 = pl.kernel(
    _agg2_body,
    out_type=[
        jax.ShapeDtypeStruct((NC, 2, NPAD), jnp.float32),
    ],
    mesh=_mesh,
    scratch_types=[
        pltpu.VMEM((NCHW, CH), jnp.int32),
        pltpu.VMEM((NCHW, CH), jnp.int32),
        pltpu.VMEM((NCHW, CH), jnp.float32),
        pltpu.VMEM((CH,), jnp.float32),
        pltpu.VMEM((CH,), jnp.float32),
        pltpu.VMEM((CH,), jnp.float32),
        pltpu.VMEM((CH,), jnp.float32),
        pltpu.VMEM((CH,), jnp.float32),
        pltpu.VMEM((CH,), jnp.float32),
        pltpu.VMEM((RPS,), jnp.float32),
        pltpu.VMEM_SHARED((NPAD,), jnp.float32),
        pltpu.VMEM_SHARED((NPAD,), jnp.float32),
        pltpu.SemaphoreType.DMA,
        pltpu.SemaphoreType.DMA,
        pltpu.SemaphoreType.DMA,
        pltpu.SemaphoreType.DMA,
    ],
)


def kernel(x, edge_index, W1, a_src1, a_dst1, b1, W2, a_src2, a_dst2, b2):
    xp = jnp.pad(x, ((0, NPAD - N), (0, 0)))
    src = edge_index[0].astype(jnp.int32)
    dst = edge_index[1].astype(jnp.int32)
    padv = jnp.full((EPAD - E,), NPAD - 1, jnp.int32)
    src2d = jnp.concatenate([src, padv]).reshape(NW * NCHW, CH)
    dst2d = jnp.concatenate([dst, padv]).reshape(NW * NCHW, CH)

    h1, as1, ad1, g1 = _tc1(xp, W1, a_src1, a_dst1)
    ex1, s1p = _edge_softmax(src2d, dst2d, as1, ad1, g1[0, :16])
    (o1a,) = _agg1(src2d, dst2d, ex1, s1p[0], s1p[1], h1[:, :DH])
    (o1b,) = _agg1(src2d, dst2d, ex1, s1p[0], s1p[1], h1[:, DH:])

    w2c0 = W2[:, 0]
    w2c1 = W2[:, 1]
    was2 = W2 @ a_src2
    wad2 = W2 @ a_dst2
    h2c, as2, ad2, g2 = _tc2(o1a, o1b, b1, w2c0, w2c1, was2, wad2)
    ex2, s2p = _edge_softmax(src2d, dst2d, as2, ad2, g2[0, :16])
    (o2p,) = _agg2(src2d, dst2d, ex2, s2p[0], s2p[1], h2c[0], h2c[1])

    o2 = o2p[0] + o2p[1]
    return o2[:, :N].T + b2


# speedup vs baseline: 21.2805x; 1.0285x over previous
import jax
import jax.numpy as jnp
from jax import lax
from jax.experimental import pallas as pl
from jax.experimental.pallas import tpu as pltpu
from jax.experimental.pallas import tpu_sc as plsc

N = 10000
E = 320000
D = 128
NPAD = 10240
NC = 2
NS = 16
NW = NC * NS
CH = 128
NCHW = 80
EPAD = NW * NCHW * CH
RPS = NPAD // NS
BLK = 256
NBLK = NPAD // BLK
NBUF = 4
DH = D // 2

_mesh = plsc.VectorSubcoreMesh(
    core_axis_name="c", subcore_axis_name="s", num_cores=NC, num_subcores=NS)


def _lrelu(v):
    return jnp.where(v >= 0.0, v, 0.2 * v)


def _tc1_body(x_ref, w_ref, asrc_ref, adst_ref,
              h_ref, as_ref, ad_ref, g_ref, m_ref):
    i = pl.program_id(0)
    h = jnp.dot(x_ref[...], w_ref[...], preferred_element_type=jnp.float32)
    h_ref[...] = h
    a1 = h @ asrc_ref[...]
    a2 = h @ adst_ref[...]
    as_ref[...] = a1
    ad_ref[...] = a2
    bm1 = jnp.max(a1)
    bm2 = jnp.max(a2)

    @pl.when(i == 0)
    def _():
        m_ref[0] = bm1
        m_ref[1] = bm2

    @pl.when(i > 0)
    def _():
        m_ref[0] = jnp.maximum(m_ref[0], bm1)
        m_ref[1] = jnp.maximum(m_ref[1], bm2)

    z = m_ref[0] + m_ref[1]
    g = jnp.where(z >= 0.0, z, 0.2 * z)
    g_ref[...] = jnp.full((8, 128), g, jnp.float32)


_tc1 = pl.pallas_call(
    _tc1_body,
    grid=(NBLK,),
    in_specs=[
        pl.BlockSpec((BLK, D), lambda i: (i, 0)),
        pl.BlockSpec((D, D), lambda i: (0, 0)),
        pl.BlockSpec((D,), lambda i: (0,)),
        pl.BlockSpec((D,), lambda i: (0,)),
    ],
    out_specs=[
        pl.BlockSpec((BLK, D), lambda i: (i, 0)),
        pl.BlockSpec((BLK,), lambda i: (i,)),
        pl.BlockSpec((BLK,), lambda i: (i,)),
        pl.BlockSpec((8, 128), lambda i: (0, 0)),
    ],
    out_shape=[
        jax.ShapeDtypeStruct((NPAD, D), jnp.float32),
        jax.ShapeDtypeStruct((NPAD,), jnp.float32),
        jax.ShapeDtypeStruct((NPAD,), jnp.float32),
        jax.ShapeDtypeStruct((8, 128), jnp.float32),
    ],
    scratch_shapes=[pltpu.SMEM((2,), jnp.float32)],
)


def _tc2_body(pa_ref, pb_ref, b1_ref, w0_ref, w1_ref, was_ref, wad_ref,
              hc_ref, as_ref, ad_ref, g_ref, m_ref):
    i = pl.program_id(0)
    hfull = jnp.concatenate([pa_ref[0] + pa_ref[1], pb_ref[0] + pb_ref[1]],
                            axis=1)
    hin = jax.nn.relu(hfull + b1_ref[...])
    c0 = hin @ w0_ref[...]
    c1 = hin @ w1_ref[...]
    hc_ref[0, :] = c0
    hc_ref[1, :] = c1
    a1 = hin @ was_ref[...]
    a2 = hin @ wad_ref[...]
    as_ref[...] = a1
    ad_ref[...] = a2
    bm1 = jnp.max(a1)
    bm2 = jnp.max(a2)

    @pl.when(i == 0)
    def _():
        m_ref[0] = bm1
        m_ref[1] = bm2

    @pl.when(i > 0)
    def _():
        m_ref[0] = jnp.maximum(m_ref[0], bm1)
        m_ref[1] = jnp.maximum(m_ref[1], bm2)

    z = m_ref[0] + m_ref[1]
    g = jnp.where(z >= 0.0, z, 0.2 * z)
    g_ref[...] = jnp.full((8, 128), g, jnp.float32)


_tc2 = pl.pallas_call(
    _tc2_body,
    grid=(NBLK,),
    in_specs=[
        pl.BlockSpec((2, BLK, D // 2), lambda i: (0, i, 0)),
        pl.BlockSpec((2, BLK, D // 2), lambda i: (0, i, 0)),
        pl.BlockSpec((D,), lambda i: (0,)),
        pl.BlockSpec((D,), lambda i: (0,)),
        pl.BlockSpec((D,), lambda i: (0,)),
        pl.BlockSpec((D,), lambda i: (0,)),
        pl.BlockSpec((D,), lambda i: (0,)),
    ],
    out_specs=[
        pl.BlockSpec((2, BLK), lambda i: (0, i)),
        pl.BlockSpec((BLK,), lambda i: (i,)),
        pl.BlockSpec((BLK,), lambda i: (i,)),
        pl.BlockSpec((8, 128), lambda i: (0, 0)),
    ],
    out_shape=[
        jax.ShapeDtypeStruct((2, NPAD), jnp.float32),
        jax.ShapeDtypeStruct((NPAD,), jnp.float32),
        jax.ShapeDtypeStruct((NPAD,), jnp.float32),
        jax.ShapeDtypeStruct((8, 128), jnp.float32),
    ],
    scratch_shapes=[pltpu.SMEM((2,), jnp.float32)],
)


def _edge_body(src_hbm, dst_hbm, as_hbm, ad_hbm, g_hbm,
               ex_hbm, sp_hbm,
               src_v, dst_v, asg4, adg4, ex_v, gbuf, zer, s_sh,
               eg0, eg1, eg2, eg3, scsem):
    egsem = (eg0, eg1, eg2, eg3)
    cid = lax.axis_index("c")
    sid = lax.axis_index("s")
    wid = sid * NC + cid
    wbase = wid * NCHW
    pltpu.sync_copy(src_hbm.at[pl.ds(wbase, NCHW)], src_v)
    pltpu.sync_copy(dst_hbm.at[pl.ds(wbase, NCHW)], dst_v)
    pltpu.sync_copy(g_hbm, gbuf)

    def zb(i, carry):
        zer[pl.ds(i * 16, 16)] = jnp.zeros((16,), jnp.float32)
        return carry
    lax.fori_loop(0, RPS // 16, zb, 0)
    pltpu.sync_copy(zer, s_sh.at[pl.ds(sid * RPS, RPS)])
    plsc.subcore_barrier()

    gl = gbuf[...]

    def gstart(j, b):
        pltpu.async_copy(as_hbm.at[src_v.at[j]], asg4.at[b], egsem[b])
        pltpu.async_copy(ad_hbm.at[dst_v.at[j]], adg4.at[b], egsem[b])

    def gwait(j, b):
        pltpu.make_async_copy(as_hbm.at[src_v.at[j]], asg4.at[b],
                              egsem[b]).wait()
        pltpu.make_async_copy(ad_hbm.at[dst_v.at[j]], adg4.at[b],
                              egsem[b]).wait()

    def sstart(j):
        pltpu.async_copy(ex_v.at[j], s_sh.at[dst_v.at[j]], scsem,
                         add=True)

    def swait(j):
        pltpu.make_async_copy(ex_v.at[j], s_sh.at[dst_v.at[j]],
                              scsem).wait()

    gstart(0, 0)
    gstart(1, 1)

    def body(i, carry):
        for b in range(NBUF):
            j = i * NBUF + b
            gwait(j, b)
            bn = (b + 2) % NBUF
            if b in (0, 1):
                gstart(j + 2, bn)
            else:
                @pl.when(i < NCHW // NBUF - 1)
                def _():
                    gstart(j + 2, bn)
            for c in range(CH // 16):
                sl = pl.ds(c * 16, 16)
                v = asg4[b, sl] + adg4[b, sl]
                ex_v[j, sl] = jnp.exp(_lrelu(v) - gl)
            sstart(j)
            if b == 3:
                @pl.when(i > 0)
                def _():
                    for bb in range(NBUF):
                        swait(i * NBUF + bb - NBUF)
        return carry
    lax.fori_loop(0, NCHW // NBUF, body, 0)

    for bb in range(NBUF):
        swait(NCHW - NBUF + bb)

    pltpu.sync_copy(ex_v, ex_hbm.at[pl.ds(wbase, NCHW)])
    plsc.subcore_barrier()
    pltpu.sync_copy(s_sh.at[pl.ds(sid * RPS, RPS)],
                    sp_hbm.at[cid, pl.ds(sid * RPS, RPS)])


_edge_softmax = pl.kernel(
    _edge_body,
    out_type=[
        jax.ShapeDtypeStruct((NW * NCHW, CH), jnp.float32),
        jax.ShapeDtypeStruct((NC, NPAD), jnp.float32),
    ],
    mesh=_mesh,
    scratch_types=[
        pltpu.VMEM((NCHW, CH), jnp.int32),
        pltpu.VMEM((NCHW, CH), jnp.int32),
        pltpu.VMEM((NBUF, CH), jnp.float32),
        pltpu.VMEM((NBUF, CH), jnp.float32),
        pltpu.VMEM((NCHW, CH), jnp.float32),
        pltpu.VMEM((16,), jnp.float32),
        pltpu.VMEM((RPS,), jnp.float32),
        pltpu.VMEM_SHARED((NPAD,), jnp.float32),
        pltpu.SemaphoreType.DMA,
        pltpu.SemaphoreType.DMA,
        pltpu.SemaphoreType.DMA,
        pltpu.SemaphoreType.DMA,
        pltpu.SemaphoreType.DMA,
    ],
)


def _agg1_body(src_hbm, dst_hbm, ex_hbm, sp0_hbm, sp1_hbm, ha_hbm, hb_hbm,
               opa_hbm, opb_hbm,
               src_v, dst_v, ex_v, coef2d, rows4, sv0_4, sv1_4,
               out_sh, gs0, gs1, gs2, gs3, ss0, ss1, ss2, ss3):
    gsem = (gs0, gs1, gs2, gs3)
    ssem = (ss0, ss1, ss2, ss3)
    cid = lax.axis_index("c")
    sid = lax.axis_index("s")
    wid = sid * NC + cid
    wbase = wid * NCHW
    pltpu.sync_copy(src_hbm.at[pl.ds(wbase, NCHW)], src_v)
    pltpu.sync_copy(dst_hbm.at[pl.ds(wbase, NCHW)], dst_v)
    pltpu.sync_copy(ex_hbm.at[pl.ds(wbase, NCHW)], ex_v)

    htabs = (ha_hbm, hb_hbm)
    optabs = (opa_hbm, opb_hbm)

    for half in range(2):
        h_hbm = htabs[half]
        op_hbm = optabs[half]

        def zb(r, carry):
            for c in range(DH // 16):
                rows4[0, r, pl.ds(c * 16, 16)] = jnp.zeros((16,),
                                                           jnp.float32)
            return carry
        lax.fori_loop(0, CH, zb, 0)
        for b in range(RPS // CH):
            pltpu.sync_copy(rows4.at[0],
                            out_sh.at[pl.ds(sid * RPS + b * CH, CH)])
        plsc.subcore_barrier()

        def gstart(j, b):
            pltpu.async_copy(h_hbm.at[src_v.at[j]], rows4.at[b], gsem[b])
            if half == 0:
                pltpu.async_copy(sp0_hbm.at[dst_v.at[j]], sv0_4.at[b],
                                 gsem[b])
                pltpu.async_copy(sp1_hbm.at[dst_v.at[j]], sv1_4.at[b],
                                 gsem[b])

        def gwait(j, b):
            pltpu.make_async_copy(h_hbm.at[src_v.at[j]], rows4.at[b],
                                  gsem[b]).wait()
            if half == 0:
                pltpu.make_async_copy(sp0_hbm.at[dst_v.at[j]], sv0_4.at[b],
                                      gsem[b]).wait()
                pltpu.make_async_copy(sp1_hbm.at[dst_v.at[j]], sv1_4.at[b],
                                      gsem[b]).wait()

        def sstart(j, b):
            pltpu.async_copy(rows4.at[b], out_sh.at[dst_v.at[j]], ssem[b],
                             add=True)

        def swait(j, b):
            pltpu.make_async_copy(rows4.at[b], out_sh.at[dst_v.at[j]],
                                  ssem[b]).wait()

        def compute(j, b):
            if half == 0:
                for c in range(CH // 16):
                    sl = pl.ds(c * 16, 16)
                    coef2d[j, sl] = ex_v[j, sl] / (
                        sv0_4[b, sl] + sv1_4[b, sl] + 1e-16)

            def sc(g, carry2):
                cv = coef2d[j, pl.ds(g * 16, 16)]
                for k in range(16):
                    cb = lax.gather(
                        cv, jnp.full((16, 1), k, jnp.int32),
                        lax.GatherDimensionNumbers(
                            offset_dims=(), collapsed_slice_dims=(0,),
                            start_index_map=(0,)),
                        (1,), mode=lax.GatherScatterMode.PROMISE_IN_BOUNDS)
                    r = g * 16 + k
                    for c in range(DH // 16):
                        sl = pl.ds(c * 16, 16)
                        rows4[b, r, sl] = rows4[b, r, sl] * cb
                return carry2
            lax.fori_loop(0, CH // 16, sc, 0)

        gstart(0, 0)
        gstart(1, 1)

        def body(i, carry):
            for b in range(NBUF):
                j = i * NBUF + b
                gwait(j, b)
                compute(j, b)
                bn = (b + 2) % NBUF
                if b in (0, 1):
                    @pl.when(i > 0)
                    def _():
                        swait(j - 2, bn)
                    gstart(j + 2, bn)
                else:
                    swait(j - 2, bn)

                    @pl.when(i < NCHW // NBUF - 1)
                    def _():
                        gstart(j + 2, bn)
                sstart(j, b)
            return carry
        lax.fori_loop(0, NCHW // NBUF, body, 0)

        swait(NCHW - 2, (NCHW - 2) % NBUF)
        swait(NCHW - 1, (NCHW - 1) % NBUF)

        plsc.subcore_barrier()
        for b in range(RPS // CH):
            r0 = sid * RPS + b * CH
            pltpu.sync_copy(out_sh.at[pl.ds(r0, CH)],
                            op_hbm.at[cid, pl.ds(r0, CH)])


_agg1 = pl.kernel(
    _agg1_body,
    out_type=[
        jax.ShapeDtypeStruct((NC, NPAD, DH), jnp.float32),
        jax.ShapeDtypeStruct((NC, NPAD, DH), jnp.float32),
    ],
    mesh=_mesh,
    compiler_params=pltpu.CompilerParams(use_tc_tiling_on_sc=False),
    scratch_types=[
        pltpu.VMEM((NCHW, CH), jnp.int32),
        pltpu.VMEM((NCHW, CH), jnp.int32),
        pltpu.VMEM((NCHW, CH), jnp.float32),
        pltpu.VMEM((NCHW, CH), jnp.float32),
        pltpu.VMEM((NBUF, CH, DH), jnp.float32),
        pltpu.VMEM((NBUF, CH), jnp.float32),
        pltpu.VMEM((NBUF, CH), jnp.float32),
        pltpu.VMEM_SHARED((NPAD, DH), jnp.float32),
        pltpu.SemaphoreType.DMA,
        pltpu.SemaphoreType.DMA,
        pltpu.SemaphoreType.DMA,
        pltpu.SemaphoreType.DMA,
        pltpu.SemaphoreType.DMA,
        pltpu.SemaphoreType.DMA,
        pltpu.SemaphoreType.DMA,
        pltpu.SemaphoreType.DMA,
    ],
)


def _agg2_body(src_hbm, dst_hbm, ex_hbm, sp0_hbm, sp1_hbm, c0_hbm, c1_hbm,
               op_hbm,
               src_v, dst_v, ex_v, sv0, sv1, c0g, c1g, y0, y1, zer,
               s0_sh, s1_sh, sem, sem2, sem3, sem4):
    cid = lax.axis_index("c")
    sid = lax.axis_index("s")
    wid = sid * NC + cid
    wbase = wid * NCHW
    pltpu.sync_copy(src_hbm.at[pl.ds(wbase, NCHW)], src_v)
    pltpu.sync_copy(dst_hbm.at[pl.ds(wbase, NCHW)], dst_v)
    pltpu.sync_copy(ex_hbm.at[pl.ds(wbase, NCHW)], ex_v)

    def zb(i, carry):
        zer[pl.ds(i * 16, 16)] = jnp.zeros((16,), jnp.float32)
        return carry
    lax.fori_loop(0, RPS // 16, zb, 0)
    pltpu.sync_copy(zer, s0_sh.at[pl.ds(sid * RPS, RPS)])
    pltpu.sync_copy(zer, s1_sh.at[pl.ds(sid * RPS, RPS)])
    plsc.subcore_barrier()

    def body(j, carry):
        cpa = pltpu.async_copy(c0_hbm.at[src_v.at[j]], c0g, sem)
        cpb = pltpu.async_copy(c1_hbm.at[src_v.at[j]], c1g, sem2)
        cp0 = pltpu.async_copy(sp0_hbm.at[dst_v.at[j]], sv0, sem3)
        cp1 = pltpu.async_copy(sp1_hbm.at[dst_v.at[j]], sv1, sem4)
        cpa.wait()
        cpb.wait()
        cp0.wait()
        cp1.wait()
        for c in range(CH // 16):
            sl = pl.ds(c * 16, 16)
            coef = ex_v[j, sl] / (sv0[sl] + sv1[sl] + 1e-16)
            y0[sl] = coef * c0g[sl]
            y1[sl] = coef * c1g[sl]
        pltpu.sync_copy(y0, s0_sh.at[dst_v.at[j]], add=True)
        pltpu.sync_copy(y1, s1_sh.at[dst_v.at[j]], add=True)
        return carry
    lax.fori_loop(0, NCHW, body, 0)

    plsc.subcore_barrier()
    pltpu.sync_copy(s0_sh.at[pl.ds(sid * RPS, RPS)],
                    op_hbm.at[cid, 0, pl.ds(sid * RPS, RPS)])
    pltpu.sync_copy(s1_sh.at[pl.ds(sid * RPS, RPS)],
                    op_hbm.at[cid, 1, pl.ds(sid * RPS, RPS)])


_agg2 = pl.kernel(
    _agg2_body,
    out_type=[
        jax.ShapeDtypeStruct((NC, 2, NPAD), jnp.float32),
    ],
    mesh=_mesh,
    scratch_types=[
        pltpu.VMEM((NCHW, CH), jnp.int32),
        pltpu.VMEM((NCHW, CH), jnp.int32),
        pltpu.VMEM((NCHW, CH), jnp.float32),
        pltpu.VMEM((CH,), jnp.float32),
        pltpu.VMEM((CH,), jnp.float32),
        pltpu.VMEM((CH,), jnp.float32),
        pltpu.VMEM((CH,), jnp.float32),
        pltpu.VMEM((CH,), jnp.float32),
        pltpu.VMEM((CH,), jnp.float32),
        pltpu.VMEM((RPS,), jnp.float32),
        pltpu.VMEM_SHARED((NPAD,), jnp.float32),
        pltpu.VMEM_SHARED((NPAD,), jnp.float32),
        pltpu.SemaphoreType.DMA,
        pltpu.SemaphoreType.DMA,
        pltpu.SemaphoreType.DMA,
        pltpu.SemaphoreType.DMA,
    ],
)


def kernel(x, edge_index, W1, a_src1, a_dst1, b1, W2, a_src2, a_dst2, b2):
    xp = jnp.pad(x, ((0, NPAD - N), (0, 0)))
    src = edge_index[0].astype(jnp.int32)
    dst = edge_index[1].astype(jnp.int32)
    padv = jnp.full((EPAD - E,), NPAD - 1, jnp.int32)
    src2d = jnp.concatenate([src, padv]).reshape(NW * NCHW, CH)
    dst2d = jnp.concatenate([dst, padv]).reshape(NW * NCHW, CH)

    h1, as1, ad1, g1 = _tc1(xp, W1, a_src1, a_dst1)
    ex1, s1p = _edge_softmax(src2d, dst2d, as1, ad1, g1[0, :16])
    o1a, o1b = _agg1(src2d, dst2d, ex1, s1p[0], s1p[1],
                     h1[:, :DH], h1[:, DH:])

    w2c0 = W2[:, 0]
    w2c1 = W2[:, 1]
    was2 = W2 @ a_src2
    wad2 = W2 @ a_dst2
    h2c, as2, ad2, g2 = _tc2(o1a, o1b, b1, w2c0, w2c1, was2, wad2)
    ex2, s2p = _edge_softmax(src2d, dst2d, as2, ad2, g2[0, :16])
    (o2p,) = _agg2(src2d, dst2d, ex2, s2p[0], s2p[1], h2c[0], h2c[1])

    o2 = o2p[0] + o2p[1]
    return o2[:, :N].T + b2


# speedup vs baseline: 22.3658x; 1.0510x over previous
import jax
import jax.numpy as jnp
from jax import lax
from jax.experimental import pallas as pl
from jax.experimental.pallas import tpu as pltpu
from jax.experimental.pallas import tpu_sc as plsc

N = 10000
E = 320000
D = 128
NPAD = 10240
NC = 2
NS = 16
NW = NC * NS
CH = 128
NCHW = 80
EPAD = NW * NCHW * CH
RPS = NPAD // NS
BLK = 256
NBLK = NPAD // BLK
NBUF = 4
DH = D // 2

_mesh = plsc.VectorSubcoreMesh(
    core_axis_name="c", subcore_axis_name="s", num_cores=NC, num_subcores=NS)


def _lrelu(v):
    return jnp.where(v >= 0.0, v, 0.2 * v)


def _tc1_body(x_ref, w_ref, asrc_ref, adst_ref,
              h_ref, as_ref, ad_ref, g_ref, m_ref):
    i = pl.program_id(0)
    h = jnp.dot(x_ref[...], w_ref[...], preferred_element_type=jnp.float32)
    h_ref[...] = h
    a1 = h @ asrc_ref[...]
    a2 = h @ adst_ref[...]
    as_ref[...] = a1
    ad_ref[...] = a2
    bm1 = jnp.max(a1)
    bm2 = jnp.max(a2)

    @pl.when(i == 0)
    def _():
        m_ref[0] = bm1
        m_ref[1] = bm2

    @pl.when(i > 0)
    def _():
        m_ref[0] = jnp.maximum(m_ref[0], bm1)
        m_ref[1] = jnp.maximum(m_ref[1], bm2)

    z = m_ref[0] + m_ref[1]
    g = jnp.where(z >= 0.0, z, 0.2 * z)
    g_ref[...] = jnp.full((8, 128), g, jnp.float32)


_tc1 = pl.pallas_call(
    _tc1_body,
    grid=(NBLK,),
    in_specs=[
        pl.BlockSpec((BLK, D), lambda i: (i, 0)),
        pl.BlockSpec((D, D), lambda i: (0, 0)),
        pl.BlockSpec((D,), lambda i: (0,)),
        pl.BlockSpec((D,), lambda i: (0,)),
    ],
    out_specs=[
        pl.BlockSpec((BLK, D), lambda i: (i, 0)),
        pl.BlockSpec((BLK,), lambda i: (i,)),
        pl.BlockSpec((BLK,), lambda i: (i,)),
        pl.BlockSpec((8, 128), lambda i: (0, 0)),
    ],
    out_shape=[
        jax.ShapeDtypeStruct((NPAD, D), jnp.float32),
        jax.ShapeDtypeStruct((NPAD,), jnp.float32),
        jax.ShapeDtypeStruct((NPAD,), jnp.float32),
        jax.ShapeDtypeStruct((8, 128), jnp.float32),
    ],
    scratch_shapes=[pltpu.SMEM((2,), jnp.float32)],
)


def _tc2_body(pa_ref, pb_ref, b1_ref, w0_ref, w1_ref, was_ref, wad_ref,
              hc_ref, as_ref, ad_ref, g_ref, m_ref):
    i = pl.program_id(0)
    hfull = jnp.concatenate([pa_ref[0] + pa_ref[1], pb_ref[0] + pb_ref[1]],
                            axis=1)
    hin = jax.nn.relu(hfull + b1_ref[...])
    c0 = hin @ w0_ref[...]
    c1 = hin @ w1_ref[...]
    hc_ref[0, :] = c0
    hc_ref[1, :] = c1
    a1 = hin @ was_ref[...]
    a2 = hin @ wad_ref[...]
    as_ref[...] = a1
    ad_ref[...] = a2
    bm1 = jnp.max(a1)
    bm2 = jnp.max(a2)

    @pl.when(i == 0)
    def _():
        m_ref[0] = bm1
        m_ref[1] = bm2

    @pl.when(i > 0)
    def _():
        m_ref[0] = jnp.maximum(m_ref[0], bm1)
        m_ref[1] = jnp.maximum(m_ref[1], bm2)

    z = m_ref[0] + m_ref[1]
    g = jnp.where(z >= 0.0, z, 0.2 * z)
    g_ref[...] = jnp.full((8, 128), g, jnp.float32)


_tc2 = pl.pallas_call(
    _tc2_body,
    grid=(NBLK,),
    in_specs=[
        pl.BlockSpec((2, BLK, D // 2), lambda i: (0, i, 0)),
        pl.BlockSpec((2, BLK, D // 2), lambda i: (0, i, 0)),
        pl.BlockSpec((D,), lambda i: (0,)),
        pl.BlockSpec((D,), lambda i: (0,)),
        pl.BlockSpec((D,), lambda i: (0,)),
        pl.BlockSpec((D,), lambda i: (0,)),
        pl.BlockSpec((D,), lambda i: (0,)),
    ],
    out_specs=[
        pl.BlockSpec((2, BLK), lambda i: (0, i)),
        pl.BlockSpec((BLK,), lambda i: (i,)),
        pl.BlockSpec((BLK,), lambda i: (i,)),
        pl.BlockSpec((8, 128), lambda i: (0, 0)),
    ],
    out_shape=[
        jax.ShapeDtypeStruct((2, NPAD), jnp.float32),
        jax.ShapeDtypeStruct((NPAD,), jnp.float32),
        jax.ShapeDtypeStruct((NPAD,), jnp.float32),
        jax.ShapeDtypeStruct((8, 128), jnp.float32),
    ],
    scratch_shapes=[pltpu.SMEM((2,), jnp.float32)],
)


def _edge_body(src_hbm, dst_hbm, as_hbm, ad_hbm, g_hbm,
               ex_hbm, sp_hbm,
               src_v, dst_v, asg4, adg4, ex_v, gbuf, zer, s_sh,
               eg0, eg1, eg2, eg3, scsem):
    egsem = (eg0, eg1, eg2, eg3)
    cid = lax.axis_index("c")
    sid = lax.axis_index("s")
    wid = sid * NC + cid
    wbase = wid * NCHW
    pltpu.sync_copy(src_hbm.at[pl.ds(wbase, NCHW)], src_v)
    pltpu.sync_copy(dst_hbm.at[pl.ds(wbase, NCHW)], dst_v)
    pltpu.sync_copy(g_hbm, gbuf)

    def zb(i, carry):
        zer[pl.ds(i * 16, 16)] = jnp.zeros((16,), jnp.float32)
        return carry
    lax.fori_loop(0, RPS // 16, zb, 0)
    pltpu.sync_copy(zer, s_sh.at[pl.ds(sid * RPS, RPS)])
    plsc.subcore_barrier()

    gl = gbuf[...]

    def gstart(j, b):
        pltpu.async_copy(as_hbm.at[src_v.at[j]], asg4.at[b], egsem[b])
        pltpu.async_copy(ad_hbm.at[dst_v.at[j]], adg4.at[b], egsem[b])

    def gwait(j, b):
        pltpu.make_async_copy(as_hbm.at[src_v.at[j]], asg4.at[b],
                              egsem[b]).wait()
        pltpu.make_async_copy(ad_hbm.at[dst_v.at[j]], adg4.at[b],
                              egsem[b]).wait()

    def sstart(j):
        pltpu.async_copy(ex_v.at[j], s_sh.at[dst_v.at[j]], scsem,
                         add=True)

    def swait(j):
        pltpu.make_async_copy(ex_v.at[j], s_sh.at[dst_v.at[j]],
                              scsem).wait()

    gstart(0, 0)
    gstart(1, 1)

    def body(i, carry):
        for b in range(NBUF):
            j = i * NBUF + b
            gwait(j, b)
            bn = (b + 2) % NBUF
            if b in (0, 1):
                gstart(j + 2, bn)
            else:
                @pl.when(i < NCHW // NBUF - 1)
                def _():
                    gstart(j + 2, bn)
            for c in range(CH // 16):
                sl = pl.ds(c * 16, 16)
                v = asg4[b, sl] + adg4[b, sl]
                ex_v[j, sl] = jnp.exp(_lrelu(v) - gl)
            sstart(j)
            if b == 3:
                @pl.when(i > 0)
                def _():
                    for bb in range(NBUF):
                        swait(i * NBUF + bb - NBUF)
        return carry
    lax.fori_loop(0, NCHW // NBUF, body, 0)

    for bb in range(NBUF):
        swait(NCHW - NBUF + bb)

    pltpu.sync_copy(ex_v, ex_hbm.at[pl.ds(wbase, NCHW)])
    plsc.subcore_barrier()
    pltpu.sync_copy(s_sh.at[pl.ds(sid * RPS, RPS)],
                    sp_hbm.at[cid, pl.ds(sid * RPS, RPS)])


_edge_softmax = pl.kernel(
    _edge_body,
    out_type=[
        jax.ShapeDtypeStruct((NW * NCHW, CH), jnp.float32),
        jax.ShapeDtypeStruct((NC, NPAD), jnp.float32),
    ],
    mesh=_mesh,
    scratch_types=[
        pltpu.VMEM((NCHW, CH), jnp.int32),
        pltpu.VMEM((NCHW, CH), jnp.int32),
        pltpu.VMEM((NBUF, CH), jnp.float32),
        pltpu.VMEM((NBUF, CH), jnp.float32),
        pltpu.VMEM((NCHW, CH), jnp.float32),
        pltpu.VMEM((16,), jnp.float32),
        pltpu.VMEM((RPS,), jnp.float32),
        pltpu.VMEM_SHARED((NPAD,), jnp.float32),
        pltpu.SemaphoreType.DMA,
        pltpu.SemaphoreType.DMA,
        pltpu.SemaphoreType.DMA,
        pltpu.SemaphoreType.DMA,
        pltpu.SemaphoreType.DMA,
    ],
)


def _agg1_body(src_hbm, dst_hbm, ex_hbm, sp0_hbm, sp1_hbm, ha_hbm, hb_hbm,
               opa_hbm, opb_hbm,
               src_v, dst_v, ex_v, coef2d, rows4, sv0_4, sv1_4,
               out_sh, gs0, gs1, gs2, gs3, ss0, ss1, ss2, ss3):
    gsem = (gs0, gs1, gs2, gs3)
    ssem = (ss0, ss1, ss2, ss3)
    cid = lax.axis_index("c")
    sid = lax.axis_index("s")
    wid = sid * NC + cid
    wbase = wid * NCHW
    pltpu.sync_copy(src_hbm.at[pl.ds(wbase, NCHW)], src_v)
    pltpu.sync_copy(dst_hbm.at[pl.ds(wbase, NCHW)], dst_v)
    pltpu.sync_copy(ex_hbm.at[pl.ds(wbase, NCHW)], ex_v)

    htabs = (ha_hbm, hb_hbm)
    optabs = (opa_hbm, opb_hbm)

    for half in range(2):
        h_hbm = htabs[half]
        op_hbm = optabs[half]

        def zb(r, carry):
            for c in range(DH // 16):
                rows4[0, r, pl.ds(c * 16, 16)] = jnp.zeros((16,),
                                                           jnp.float32)
            return carry
        lax.fori_loop(0, CH, zb, 0)
        for b in range(RPS // CH):
            pltpu.sync_copy(rows4.at[0],
                            out_sh.at[pl.ds(sid * RPS + b * CH, CH)])
        plsc.subcore_barrier()

        def gstart(j, b):
            pltpu.async_copy(h_hbm.at[src_v.at[j]], rows4.at[b], gsem[b])
            if half == 0:
                pltpu.async_copy(sp0_hbm.at[dst_v.at[j]], sv0_4.at[b],
                                 gsem[b])
                pltpu.async_copy(sp1_hbm.at[dst_v.at[j]], sv1_4.at[b],
                                 gsem[b])

        def gwait(j, b):
            pltpu.make_async_copy(h_hbm.at[src_v.at[j]], rows4.at[b],
                                  gsem[b]).wait()
            if half == 0:
                pltpu.make_async_copy(sp0_hbm.at[dst_v.at[j]], sv0_4.at[b],
                                      gsem[b]).wait()
                pltpu.make_async_copy(sp1_hbm.at[dst_v.at[j]], sv1_4.at[b],
                                      gsem[b]).wait()

        def sstart(j, b):
            pltpu.async_copy(rows4.at[b], out_sh.at[dst_v.at[j]], ssem[b],
                             add=True)

        def swait(j, b):
            pltpu.make_async_copy(rows4.at[b], out_sh.at[dst_v.at[j]],
                                  ssem[b]).wait()

        def compute(j, b):
            if half == 0:
                for c in range(CH // 16):
                    sl = pl.ds(c * 16, 16)
                    coef2d[j, sl] = ex_v[j, sl] / (
                        sv0_4[b, sl] + sv1_4[b, sl] + 1e-16)

            def sc(g, carry2):
                cv = coef2d[j, pl.ds(g * 16, 16)]
                for k in range(16):
                    cb = lax.gather(
                        cv, jnp.full((16, 1), k, jnp.int32),
                        lax.GatherDimensionNumbers(
                            offset_dims=(), collapsed_slice_dims=(0,),
                            start_index_map=(0,)),
                        (1,), mode=lax.GatherScatterMode.PROMISE_IN_BOUNDS)
                    r = g * 16 + k
                    for c in range(DH // 16):
                        sl = pl.ds(c * 16, 16)
                        rows4[b, r, sl] = rows4[b, r, sl] * cb
                return carry2
            lax.fori_loop(0, CH // 16, sc, 0)

        gstart(0, 0)
        gstart(1, 1)

        def body(i, carry):
            for b in range(NBUF):
                j = i * NBUF + b
                gwait(j, b)
                compute(j, b)
                bn = (b + 2) % NBUF
                if b in (0, 1):
                    @pl.when(i > 0)
                    def _():
                        swait(j - 2, bn)
                    gstart(j + 2, bn)
                else:
                    swait(j - 2, bn)

                    @pl.when(i < NCHW // NBUF - 1)
                    def _():
                        gstart(j + 2, bn)
                sstart(j, b)
            return carry
        lax.fori_loop(0, NCHW // NBUF, body, 0)

        swait(NCHW - 2, (NCHW - 2) % NBUF)
        swait(NCHW - 1, (NCHW - 1) % NBUF)

        plsc.subcore_barrier()
        for b in range(RPS // CH):
            r0 = sid * RPS + b * CH
            pltpu.sync_copy(out_sh.at[pl.ds(r0, CH)],
                            op_hbm.at[cid, pl.ds(r0, CH)])


_agg1 = pl.kernel(
    _agg1_body,
    out_type=[
        jax.ShapeDtypeStruct((NC, NPAD, DH), jnp.float32),
        jax.ShapeDtypeStruct((NC, NPAD, DH), jnp.float32),
    ],
    mesh=_mesh,
    compiler_params=pltpu.CompilerParams(use_tc_tiling_on_sc=False),
    scratch_types=[
        pltpu.VMEM((NCHW, CH), jnp.int32),
        pltpu.VMEM((NCHW, CH), jnp.int32),
        pltpu.VMEM((NCHW, CH), jnp.float32),
        pltpu.VMEM((NCHW, CH), jnp.float32),
        pltpu.VMEM((NBUF, CH, DH), jnp.float32),
        pltpu.VMEM((NBUF, CH), jnp.float32),
        pltpu.VMEM((NBUF, CH), jnp.float32),
        pltpu.VMEM_SHARED((NPAD, DH), jnp.float32),
        pltpu.SemaphoreType.DMA,
        pltpu.SemaphoreType.DMA,
        pltpu.SemaphoreType.DMA,
        pltpu.SemaphoreType.DMA,
        pltpu.SemaphoreType.DMA,
        pltpu.SemaphoreType.DMA,
        pltpu.SemaphoreType.DMA,
        pltpu.SemaphoreType.DMA,
    ],
)


def _agg2_body(src_hbm, dst_hbm, ex_hbm, sp0_hbm, sp1_hbm, c0_hbm, c1_hbm,
               op_hbm,
               src_v, dst_v, ex_v, sv0_4, sv1_4, c0g4, c1g4, y0_4, y1_4, zer,
               s0_sh, s1_sh, gs0, gs1, gs2, gs3, ss0, ss1, ss2, ss3):
    gsem = (gs0, gs1, gs2, gs3)
    ssem = (ss0, ss1, ss2, ss3)
    cid = lax.axis_index("c")
    sid = lax.axis_index("s")
    wid = sid * NC + cid
    wbase = wid * NCHW
    pltpu.sync_copy(src_hbm.at[pl.ds(wbase, NCHW)], src_v)
    pltpu.sync_copy(dst_hbm.at[pl.ds(wbase, NCHW)], dst_v)
    pltpu.sync_copy(ex_hbm.at[pl.ds(wbase, NCHW)], ex_v)

    def zb(i, carry):
        zer[pl.ds(i * 16, 16)] = jnp.zeros((16,), jnp.float32)
        return carry
    lax.fori_loop(0, RPS // 16, zb, 0)
    pltpu.sync_copy(zer, s0_sh.at[pl.ds(sid * RPS, RPS)])
    pltpu.sync_copy(zer, s1_sh.at[pl.ds(sid * RPS, RPS)])
    plsc.subcore_barrier()

    def gstart(j, b):
        pltpu.async_copy(c0_hbm.at[src_v.at[j]], c0g4.at[b], gsem[b])
        pltpu.async_copy(c1_hbm.at[src_v.at[j]], c1g4.at[b], gsem[b])
        pltpu.async_copy(sp0_hbm.at[dst_v.at[j]], sv0_4.at[b], gsem[b])
        pltpu.async_copy(sp1_hbm.at[dst_v.at[j]], sv1_4.at[b], gsem[b])

    def gwait(j, b):
        pltpu.make_async_copy(c0_hbm.at[src_v.at[j]], c0g4.at[b],
                              gsem[b]).wait()
        pltpu.make_async_copy(c1_hbm.at[src_v.at[j]], c1g4.at[b],
                              gsem[b]).wait()
        pltpu.make_async_copy(sp0_hbm.at[dst_v.at[j]], sv0_4.at[b],
                              gsem[b]).wait()
        pltpu.make_async_copy(sp1_hbm.at[dst_v.at[j]], sv1_4.at[b],
                              gsem[b]).wait()

    def sstart(j, b):
        pltpu.async_copy(y0_4.at[b], s0_sh.at[dst_v.at[j]], ssem[b],
                         add=True)
        pltpu.async_copy(y1_4.at[b], s1_sh.at[dst_v.at[j]], ssem[b],
                         add=True)

    def swait(j, b):
        pltpu.make_async_copy(y0_4.at[b], s0_sh.at[dst_v.at[j]],
                              ssem[b]).wait()
        pltpu.make_async_copy(y1_4.at[b], s1_sh.at[dst_v.at[j]],
                              ssem[b]).wait()

    gstart(0, 0)
    gstart(1, 1)

    def body(i, carry):
        for b in range(NBUF):
            j = i * NBUF + b
            gwait(j, b)
            bn = (b + 2) % NBUF
            if b in (0, 1):
                @pl.when(i > 0)
                def _():
                    swait(j - 2, bn)
                gstart(j + 2, bn)
            else:
                swait(j - 2, bn)

                @pl.when(i < NCHW // NBUF - 1)
                def _():
                    gstart(j + 2, bn)
            for c in range(CH // 16):
                sl = pl.ds(c * 16, 16)
                coef = ex_v[j, sl] / (sv0_4[b, sl] + sv1_4[b, sl] + 1e-16)
                y0_4[b, sl] = coef * c0g4[b, sl]
                y1_4[b, sl] = coef * c1g4[b, sl]
            sstart(j, b)
        return carry
    lax.fori_loop(0, NCHW // NBUF, body, 0)

    swait(NCHW - 2, (NCHW - 2) % NBUF)
    swait(NCHW - 1, (NCHW - 1) % NBUF)

    plsc.subcore_barrier()
    pltpu.sync_copy(s0_sh.at[pl.ds(sid * RPS, RPS)],
                    op_hbm.at[cid, 0, pl.ds(sid * RPS, RPS)])
    pltpu.sync_copy(s1_sh.at[pl.ds(sid * RPS, RPS)],
                    op_hbm.at[cid, 1, pl.ds(sid * RPS, RPS)])


_agg2 = pl.kernel(
    _agg2_body,
    out_type=[
        jax.ShapeDtypeStruct((NC, 2, NPAD), jnp.float32),
    ],
    mesh=_mesh,
    scratch_types=[
        pltpu.VMEM((NCHW, CH), jnp.int32),
        pltpu.VMEM((NCHW, CH), jnp.int32),
        pltpu.VMEM((NCHW, CH), jnp.float32),
        pltpu.VMEM((NBUF, CH), jnp.float32),
        pltpu.VMEM((NBUF, CH), jnp.float32),
        pltpu.VMEM((NBUF, CH), jnp.float32),
        pltpu.VMEM((NBUF, CH), jnp.float32),
        pltpu.VMEM((NBUF, CH), jnp.float32),
        pltpu.VMEM((NBUF, CH), jnp.float32),
        pltpu.VMEM((RPS,), jnp.float32),
        pltpu.VMEM_SHARED((NPAD,), jnp.float32),
        pltpu.VMEM_SHARED((NPAD,), jnp.float32),
        pltpu.SemaphoreType.DMA,
        pltpu.SemaphoreType.DMA,
        pltpu.SemaphoreType.DMA,
        pltpu.SemaphoreType.DMA,
        pltpu.SemaphoreType.DMA,
        pltpu.SemaphoreType.DMA,
        pltpu.SemaphoreType.DMA,
        pltpu.SemaphoreType.DMA,
    ],
)


def kernel(x, edge_index, W1, a_src1, a_dst1, b1, W2, a_src2, a_dst2, b2):
    xp = jnp.pad(x, ((0, NPAD - N), (0, 0)))
    src = edge_index[0].astype(jnp.int32)
    dst = edge_index[1].astype(jnp.int32)
    padv = jnp.full((EPAD - E,), NPAD - 1, jnp.int32)
    src2d = jnp.concatenate([src, padv]).reshape(NW * NCHW, CH)
    dst2d = jnp.concatenate([dst, padv]).reshape(NW * NCHW, CH)

    h1, as1, ad1, g1 = _tc1(xp, W1, a_src1, a_dst1)
    ex1, s1p = _edge_softmax(src2d, dst2d, as1, ad1, g1[0, :16])
    o1a, o1b = _agg1(src2d, dst2d, ex1, s1p[0], s1p[1],
                     h1[:, :DH], h1[:, DH:])

    w2c0 = W2[:, 0]
    w2c1 = W2[:, 1]
    was2 = W2 @ a_src2
    wad2 = W2 @ a_dst2
    h2c, as2, ad2, g2 = _tc2(o1a, o1b, b1, w2c0, w2c1, was2, wad2)
    ex2, s2p = _edge_softmax(src2d, dst2d, as2, ad2, g2[0, :16])
    (o2p,) = _agg2(src2d, dst2d, ex2, s2p[0], s2p[1], h2c[0], h2c[1])

    o2 = o2p[0] + o2p[1]
    return o2[:, :N].T + b2
